# baseline (speedup 1.0000x reference)
import jax
import jax.numpy as jnp
import numpy as np
from jax.experimental import pallas as pl
from jax.experimental.pallas import tpu as pltpu

B, S, D_MODEL = 4, 8192, 1024
D_ROUTING = 256
N_INPUT = 32768
K = 2048
S_CHUNK = 1024
N_CHUNK = 4096
NCH = 16
CH = 2048

_INT_MIN = np.int32(-2147483648)
_M7F = np.int32(0x7FFFFFFF)


def _maxpool_body(x_ref, o_ref):
    s = pl.program_id(1)
    m = jnp.max(x_ref[0], axis=0, keepdims=True)[None]

    @pl.when(s == 0)
    def _init():
        o_ref[...] = m

    @pl.when(s != 0)
    def _acc():
        o_ref[...] = jnp.maximum(o_ref[...], m)


def _maxpool(x):
    out = pl.pallas_call(
        _maxpool_body,
        grid=(B, S // S_CHUNK),
        in_specs=[pl.BlockSpec((1, S_CHUNK, D_MODEL), lambda b, s: (b, s, 0))],
        out_specs=pl.BlockSpec((1, 1, D_MODEL), lambda b, s: (b, 0, 0)),
        out_shape=jax.ShapeDtypeStruct((B, 1, D_MODEL), jnp.float32),
        compiler_params=pltpu.CompilerParams(
            dimension_semantics=("parallel", "arbitrary")),
    )(x)
    return out.reshape(B, D_MODEL)


def _logits_body(q_ref, nk_ref, o_ref):
    res = jax.lax.dot_general(
        q_ref[...], nk_ref[...],
        dimension_numbers=(((1,), (1,)), ((), ())),
        preferred_element_type=jnp.float32,
    ) * 0.0625
    o_ref[0] = res[:, :CH]
    o_ref[1] = res[:, CH:]


def _logits(query, nk):
    return pl.pallas_call(
        _logits_body,
        grid=(N_INPUT // N_CHUNK,),
        in_specs=[pl.BlockSpec((B, D_ROUTING), lambda n: (0, 0)),
                  pl.BlockSpec((N_CHUNK, D_ROUTING), lambda n: (n, 0))],
        out_specs=pl.BlockSpec((2, B, CH), lambda n: (n, 0, 0)),
        out_shape=jax.ShapeDtypeStruct((NCH, B, CH), jnp.float32),
    )(query, nk)


def _select_body(v_ref, idx_ref, oh_ref, m_ref, key_ref, pos_ref,
                 cv_ref, ci_ref, rank_ref):
    f32 = jnp.float32
    for cc in range(8):
        qi = jax.lax.broadcasted_iota(jnp.int32, (CH, 256), 0)
        pi = jax.lax.broadcasted_iota(jnp.int32, (CH, 256), 1) + cc * 256
        m_ref[:, cc * 256:(cc + 1) * 256] = (qi < pi).astype(jnp.bfloat16)

    v = v_ref[...]
    bits = jax.lax.bitcast_convert_type(v, jnp.int32)
    key = bits ^ (jax.lax.shift_right_arithmetic(bits, 31) & _M7F)
    key_ref[...] = key

    def _count_ge(t):
        m = (key >= t[None]).astype(f32)
        return jnp.sum(jnp.sum(m, axis=2), axis=0)[:, None]

    kf = np.float32(K)
    T0 = jnp.where(_count_ge(jnp.zeros((B, 1), jnp.int32)) >= kf,
                   np.int32(0), _INT_MIN)

    def _bs_body(it, T):
        cand = T | (np.int32(1) << (np.int32(30) - it))
        return jnp.where(_count_ge(cand) >= kf, cand, T)

    T = jax.lax.fori_loop(0, 31, _bs_body, T0)

    cnt_gt = _count_ge(T) - jnp.sum(
        jnp.sum((key == T[None]).astype(f32), axis=2), axis=0)[:, None]
    need = kf - cnt_gt

    M = m_ref[...]
    dn = (((1,), (0,)), ((), ()))

    def _cs_body(c, bases):
        base_eq, base_sel = bases
        kc = key_ref[c]
        eq_c = kc == T
        gt_c = kc > T
        ecs_eq = jax.lax.dot_general(
            eq_c.astype(jnp.bfloat16), M, dn,
            preferred_element_type=f32) + base_eq
        sel_c = gt_c | (eq_c & (ecs_eq < need))
        sel_f = sel_c.astype(f32)
        pos_c = jax.lax.dot_general(
            sel_c.astype(jnp.bfloat16), M, dn,
            preferred_element_type=f32) + base_sel
        oh_ref[c] = sel_f
        pos_ref[c] = pos_c
        return (base_eq + jnp.sum(eq_c.astype(f32), axis=1, keepdims=True),
                base_sel + jnp.sum(sel_f, axis=1, keepdims=True))

    jax.lax.fori_loop(
        0, NCH, _cs_body,
        (jnp.zeros((B, 1), f32), jnp.zeros((B, 1), f32)))

    h_col = jax.lax.broadcasted_iota(jnp.int32, (16, 1), 0)
    l_col = jax.lax.broadcasted_iota(jnp.int32, (128, 1), 0)
    dnc = (((1,), (1,)), ((), ()))
    hp = jax.lax.Precision.HIGHEST
    iota_l = jax.lax.broadcasted_iota(jnp.int32, (1, CH), 1)

    for b in range(B):
        def _cp_body(c, acc, _b=b):
            cv, ci = acc
            pos_c = pos_ref[c][_b:_b + 1]
            sel_c = oh_ref[c][_b:_b + 1]
            v_c = v_ref[c][_b:_b + 1]
            pci = pos_c.astype(jnp.int32)
            hi_s = jax.lax.shift_right_logical(pci, 7)
            lo_s = pci & np.int32(127)
            jx = (iota_l + c * CH).astype(f32)
            a_m = (h_col == hi_s).astype(f32) * sel_c
            b_m = l_col == lo_s
            b_v = jnp.where(b_m, v_c, np.float32(0.0))
            b_i = jnp.where(b_m, jx, np.float32(0.0))
            cv = cv + jax.lax.dot_general(a_m, b_v, dnc, precision=hp,
                                          preferred_element_type=f32)
            ci = ci + jax.lax.dot_general(a_m, b_i, dnc, precision=hp,
                                          preferred_element_type=f32)
            return (cv, ci)

        cv, ci = jax.lax.fori_loop(
            0, NCH, _cp_body,
            (jnp.zeros((16, 128), f32), jnp.zeros((16, 128), f32)))
        cv_ref[...] = cv[:, None, :]
        ci_ref[...] = ci[:, None, :]

        cv3 = cv[:, :, None]
        ci3 = ci[:, :, None]

        def _rk_body(qs, rank):
            vq = cv_ref[qs][None]
            iq = ci_ref[qs][None]
            beats = (vq > cv3) | ((vq == cv3) & (iq < ci3))
            return rank + jnp.sum(beats.astype(f32), axis=2)

        rank = jax.lax.fori_loop(0, 16, _rk_body,
                                 jnp.zeros((16, 128), f32))
        rank_ref[...] = rank[:, None, :]

        def _pm_body(ps, out):
            rrow = rank_ref[ps].astype(jnp.int32)
            cirow = ci_ref[ps]
            rhi = jax.lax.shift_right_logical(rrow, 7)
            rlo = rrow & np.int32(127)
            ra = (h_col == rhi).astype(f32)
            rbm = l_col == rlo
            rbi = jnp.where(rbm, cirow, np.float32(0.0))
            return out + jax.lax.dot_general(ra, rbi, dnc, precision=hp,
                                             preferred_element_type=f32)

        out = jax.lax.fori_loop(0, 16, _pm_body,
                                jnp.zeros((16, 128), f32))
        idx_ref[b] = out.astype(jnp.int32)


def _select(logits3):
    idx, oh = pl.pallas_call(
        _select_body,
        out_shape=[jax.ShapeDtypeStruct((B, 16, 128), jnp.int32),
                   jax.ShapeDtypeStruct((NCH, B, CH), jnp.float32)],
        scratch_shapes=[pltpu.VMEM((CH, CH), jnp.bfloat16),
                        pltpu.VMEM((NCH, B, CH), jnp.int32),
                        pltpu.VMEM((NCH, B, CH), jnp.float32),
                        pltpu.VMEM((16, 1, 128), jnp.float32),
                        pltpu.VMEM((16, 1, 128), jnp.float32),
                        pltpu.VMEM((16, 1, 128), jnp.float32)],
    )(logits3)
    input_idx = idx.reshape(B, K)
    one_hot = oh.transpose(1, 0, 2).reshape(B, N_INPUT)
    return input_idx, one_hot


def kernel(x, W1, b1, ln_g, ln_b, W2, b2, neuron_keys, k_input):
    gc = _maxpool(x)
    h = gc @ W1 + b1
    h = jax.nn.gelu(h, approximate=False)
    mu = jnp.mean(h, axis=-1, keepdims=True)
    var = jnp.mean((h - mu) ** 2, axis=-1, keepdims=True)
    h = (h - mu) / jnp.sqrt(var + 1e-5) * ln_g + ln_b
    query = h @ W2 + b2
    logits3 = _logits(query, neuron_keys)
    input_idx, one_hot = _select(logits3)
    return (input_idx, one_hot)


# speedup vs baseline: 1.2504x; 1.2504x over previous
import jax
import jax.numpy as jnp
import numpy as np
from jax.experimental import pallas as pl
from jax.experimental.pallas import tpu as pltpu

B, S, D_MODEL = 4, 8192, 1024
D_ROUTING = 256
N_INPUT = 32768
K = 2048
S_CHUNK = 1024
N_CHUNK = 4096
LOGN = 15

_M7F = np.int32(0x7FFFFFFF)


def _maxpool_body(x_ref, o_ref):
    s = pl.program_id(1)
    m = jnp.max(x_ref[0], axis=0, keepdims=True)[None]

    @pl.when(s == 0)
    def _init():
        o_ref[...] = m

    @pl.when(s != 0)
    def _acc():
        o_ref[...] = jnp.maximum(o_ref[...], m)


def _maxpool(x):
    out = pl.pallas_call(
        _maxpool_body,
        grid=(B, S // S_CHUNK),
        in_specs=[pl.BlockSpec((1, S_CHUNK, D_MODEL), lambda b, s: (b, s, 0))],
        out_specs=pl.BlockSpec((1, 1, D_MODEL), lambda b, s: (b, 0, 0)),
        out_shape=jax.ShapeDtypeStruct((B, 1, D_MODEL), jnp.float32),
        compiler_params=pltpu.CompilerParams(
            dimension_semantics=("parallel", "arbitrary")),
    )(x)
    return out.reshape(B, D_MODEL)


def _logits_body(q_ref, nk_ref, o_ref):
    o_ref[...] = jax.lax.dot_general(
        q_ref[...], nk_ref[...],
        dimension_numbers=(((1,), (1,)), ((), ())),
        preferred_element_type=jnp.float32,
    ) * 0.0625


def _logits(query, nk):
    return pl.pallas_call(
        _logits_body,
        grid=(N_INPUT // N_CHUNK,),
        in_specs=[pl.BlockSpec((B, D_ROUTING), lambda n: (0, 0)),
                  pl.BlockSpec((N_CHUNK, D_ROUTING), lambda n: (n, 0))],
        out_specs=pl.BlockSpec((B, N_CHUNK), lambda n: (0, n)),
        out_shape=jax.ShapeDtypeStruct((B, N_INPUT), jnp.float32),
    )(query, nk)


def _rolled(x, d):
    d = d % N_INPUT
    if d == 0:
        return x
    return jnp.concatenate([x[:, d:], x[:, :d]], axis=1)


def _select_body(v_ref, idx_ref, oh_ref, k_ref, i_ref):
    v = v_ref[...]
    bits = jax.lax.bitcast_convert_type(v, jnp.int32)
    key0 = bits ^ (jax.lax.shift_right_arithmetic(bits, 31) & _M7F)
    li = jax.lax.broadcasted_iota(jnp.int32, (B, N_INPUT), 1)
    k_ref[...] = key0
    i_ref[...] = li

    for kk in range(1, LOGN + 1):
        for j in range(kk - 1, -1, -1):
            d = 1 << j
            kcur = k_ref[...]
            icur = i_ref[...]
            kup = _rolled(kcur, d)
            kdn = _rolled(kcur, N_INPUT - d)
            iup = _rolled(icur, d)
            idn = _rolled(icur, N_INPUT - d)
            low = (li & d) == 0
            kp = jnp.where(low, kup, kdn)
            ip = jnp.where(low, iup, idn)
            asc = (li & (1 << kk)) == 0
            pref = (kcur > kp) | ((kcur == kp) & (icur < ip))
            keep = (low == asc) == pref
            k_ref[...] = jnp.where(keep, kcur, kp)
            i_ref[...] = jnp.where(keep, icur, ip)

    ksort = k_ref[...]
    isort = i_ref[...]
    idx_ref[...] = isort[:, :K]
    tk = ksort[:, K - 1:K]
    ti = isort[:, K - 1:K]
    sel = (key0 > tk) | ((key0 == tk) & (li <= ti))
    oh_ref[...] = sel.astype(jnp.float32)


def _select(logits):
    idx, oh = pl.pallas_call(
        _select_body,
        out_shape=[jax.ShapeDtypeStruct((B, K), jnp.int32),
                   jax.ShapeDtypeStruct((B, N_INPUT), jnp.float32)],
        scratch_shapes=[pltpu.VMEM((B, N_INPUT), jnp.int32),
                        pltpu.VMEM((B, N_INPUT), jnp.int32)],
    )(logits)
    return idx, oh


def kernel(x, W1, b1, ln_g, ln_b, W2, b2, neuron_keys, k_input):
    gc = _maxpool(x)
    h = gc @ W1 + b1
    h = jax.nn.gelu(h, approximate=False)
    mu = jnp.mean(h, axis=-1, keepdims=True)
    var = jnp.mean((h - mu) ** 2, axis=-1, keepdims=True)
    h = (h - mu) / jnp.sqrt(var + 1e-5) * ln_g + ln_b
    query = h @ W2 + b2
    logits = _logits(query, neuron_keys)
    input_idx, one_hot = _select(logits)
    return (input_idx, one_hot)


# speedup vs baseline: 1.2688x; 1.0147x over previous
import jax
import jax.numpy as jnp
import numpy as np
from jax.experimental import pallas as pl
from jax.experimental.pallas import tpu as pltpu

B, S, D_MODEL = 4, 8192, 1024
D_ROUTING = 256
N_INPUT = 32768
K = 2048
S_CHUNK = 1024
N_CHUNK = 4096
LOGN = 15

_M7F = np.int32(0x7FFFFFFF)


def _maxpool_body(x_ref, o_ref):
    s = pl.program_id(1)
    m = jnp.max(x_ref[0], axis=0, keepdims=True)[None]

    @pl.when(s == 0)
    def _init():
        o_ref[...] = m

    @pl.when(s != 0)
    def _acc():
        o_ref[...] = jnp.maximum(o_ref[...], m)


def _maxpool(x):
    out = pl.pallas_call(
        _maxpool_body,
        grid=(B, S // S_CHUNK),
        in_specs=[pl.BlockSpec((1, S_CHUNK, D_MODEL), lambda b, s: (b, s, 0))],
        out_specs=pl.BlockSpec((1, 1, D_MODEL), lambda b, s: (b, 0, 0)),
        out_shape=jax.ShapeDtypeStruct((B, 1, D_MODEL), jnp.float32),
        compiler_params=pltpu.CompilerParams(
            dimension_semantics=("parallel", "arbitrary")),
    )(x)
    return out.reshape(B, D_MODEL)


def _logits_body(q_ref, nk_ref, o_ref):
    o_ref[...] = jax.lax.dot_general(
        q_ref[...], nk_ref[...],
        dimension_numbers=(((1,), (1,)), ((), ())),
        preferred_element_type=jnp.float32,
    ) * 0.0625


def _logits(query, nk):
    return pl.pallas_call(
        _logits_body,
        grid=(N_INPUT // N_CHUNK,),
        in_specs=[pl.BlockSpec((B, D_ROUTING), lambda n: (0, 0)),
                  pl.BlockSpec((N_CHUNK, D_ROUTING), lambda n: (n, 0))],
        out_specs=pl.BlockSpec((B, N_CHUNK), lambda n: (0, n)),
        out_shape=jax.ShapeDtypeStruct((B, N_INPUT), jnp.float32),
    )(query, nk)


def _rolled(x, d):
    d = d % N_INPUT
    if d == 0:
        return x
    return pltpu.roll(x, N_INPUT - d, 1)


def _select_body(v_ref, idx_ref, oh_ref):
    v = v_ref[...]
    bits = jax.lax.bitcast_convert_type(v, jnp.int32)
    key0 = bits ^ (jax.lax.shift_right_arithmetic(bits, 31) & _M7F)
    li = jax.lax.broadcasted_iota(jnp.int32, (B, N_INPUT), 1)
    kcur = key0
    icur = li

    for kk in range(1, LOGN + 1):
        for j in range(kk - 1, -1, -1):
            d = 1 << j
            kup = _rolled(kcur, d)
            kdn = _rolled(kcur, N_INPUT - d)
            iup = _rolled(icur, d)
            idn = _rolled(icur, N_INPUT - d)
            low = (li & d) == 0
            kp = jnp.where(low, kup, kdn)
            ip = jnp.where(low, iup, idn)
            asc = (li & (1 << kk)) == 0
            pref = (kcur > kp) | ((kcur == kp) & (icur < ip))
            keep = (low == asc) == pref
            kcur = jnp.where(keep, kcur, kp)
            icur = jnp.where(keep, icur, ip)

    ksort = kcur
    isort = icur
    idx_ref[...] = isort[:, :K]
    tk = ksort[:, K - 1:K]
    ti = isort[:, K - 1:K]
    sel = (key0 > tk) | ((key0 == tk) & (li <= ti))
    oh_ref[...] = sel.astype(jnp.float32)


def _select(logits):
    idx, oh = pl.pallas_call(
        _select_body,
        out_shape=[jax.ShapeDtypeStruct((B, K), jnp.int32),
                   jax.ShapeDtypeStruct((B, N_INPUT), jnp.float32)],
    )(logits)
    return idx, oh


def kernel(x, W1, b1, ln_g, ln_b, W2, b2, neuron_keys, k_input):
    gc = _maxpool(x)
    h = gc @ W1 + b1
    h = jax.nn.gelu(h, approximate=False)
    mu = jnp.mean(h, axis=-1, keepdims=True)
    var = jnp.mean((h - mu) ** 2, axis=-1, keepdims=True)
    h = (h - mu) / jnp.sqrt(var + 1e-5) * ln_g + ln_b
    query = h @ W2 + b2
    logits = _logits(query, neuron_keys)
    input_idx, one_hot = _select(logits)
    return (input_idx, one_hot)


# speedup vs baseline: 1.7944x; 1.4143x over previous
import jax
import jax.numpy as jnp
import numpy as np
from jax.experimental import pallas as pl
from jax.experimental.pallas import tpu as pltpu

B, S, D_MODEL = 4, 8192, 1024
D_ROUTING = 256
N_INPUT = 32768
K = 2048
S_CHUNK = 1024
N_CHUNK = 4096
LOGN = 15

_M7F = np.int32(0x7FFFFFFF)


def _maxpool_body(x_ref, o_ref):
    s = pl.program_id(1)
    m = jnp.max(x_ref[0], axis=0, keepdims=True)[None]

    @pl.when(s == 0)
    def _init():
        o_ref[...] = m

    @pl.when(s != 0)
    def _acc():
        o_ref[...] = jnp.maximum(o_ref[...], m)


def _maxpool(x):
    out = pl.pallas_call(
        _maxpool_body,
        grid=(B, S // S_CHUNK),
        in_specs=[pl.BlockSpec((1, S_CHUNK, D_MODEL), lambda b, s: (b, s, 0))],
        out_specs=pl.BlockSpec((1, 1, D_MODEL), lambda b, s: (b, 0, 0)),
        out_shape=jax.ShapeDtypeStruct((B, 1, D_MODEL), jnp.float32),
        compiler_params=pltpu.CompilerParams(
            dimension_semantics=("parallel", "arbitrary")),
    )(x)
    return out.reshape(B, D_MODEL)


def _logits_body(q_ref, nk_ref, o_ref):
    o_ref[...] = jax.lax.dot_general(
        q_ref[...], nk_ref[...],
        dimension_numbers=(((1,), (1,)), ((), ())),
        preferred_element_type=jnp.float32,
    ) * 0.0625


def _logits(query, nk):
    return pl.pallas_call(
        _logits_body,
        grid=(N_INPUT // N_CHUNK,),
        in_specs=[pl.BlockSpec((B, D_ROUTING), lambda n: (0, 0)),
                  pl.BlockSpec((N_CHUNK, D_ROUTING), lambda n: (n, 0))],
        out_specs=pl.BlockSpec((B, N_CHUNK), lambda n: (0, n)),
        out_shape=jax.ShapeDtypeStruct((B, N_INPUT), jnp.float32),
    )(query, nk)


NW = 64
CHW = 2048
LOGC = 11


def _select_body(v_ref, idx_ref, oh_ref):
    v = v_ref[...]
    bits = jax.lax.bitcast_convert_type(v, jnp.int32)
    key0 = bits ^ (jax.lax.shift_right_arithmetic(bits, 31) & _M7F)
    ti_ = jax.lax.broadcasted_iota(jnp.int32, (NW, CHW), 1)
    ci_ = jax.lax.broadcasted_iota(jnp.int32, (NW, CHW), 0) & np.int32(15)
    li = ci_ * np.int32(CHW) + ti_
    kcur = key0
    icur = li

    for kk in range(1, LOGN + 1):
        for j in range(kk - 1, -1, -1):
            if j < LOGC:
                d = 1 << j
                kup = pltpu.roll(kcur, CHW - d, 1)
                kdn = pltpu.roll(kcur, d, 1)
                iup = pltpu.roll(icur, CHW - d, 1)
                idn = pltpu.roll(icur, d, 1)
                low = (ti_ & np.int32(d)) == 0
            else:
                dc = 1 << (j - LOGC)
                kup = pltpu.roll(kcur, NW - dc, 0)
                kdn = pltpu.roll(kcur, dc, 0)
                iup = pltpu.roll(icur, NW - dc, 0)
                idn = pltpu.roll(icur, dc, 0)
                low = (ci_ & np.int32(dc)) == 0
            kp = jnp.where(low, kup, kdn)
            ip = jnp.where(low, iup, idn)
            if kk < LOGC:
                asc = (ti_ & np.int32(1 << kk)) == 0
            else:
                asc = (ci_ & np.int32(1 << (kk - LOGC))) == 0
            pref = (kcur > kp) | ((kcur == kp) & (icur < ip))
            keep = (low == asc) == pref
            kcur = jnp.where(keep, kcur, kp)
            icur = jnp.where(keep, icur, ip)

    idx_ref[...] = jnp.concatenate(
        [icur[16 * b2:16 * b2 + 1, :] for b2 in range(B)], axis=0)
    tk = jnp.concatenate(
        [jnp.broadcast_to(kcur[16 * b2:16 * b2 + 1, K - 1:K], (16, 1))
         for b2 in range(B)], axis=0)
    tif = jnp.concatenate(
        [jnp.broadcast_to(icur[16 * b2:16 * b2 + 1, K - 1:K], (16, 1))
         for b2 in range(B)], axis=0)
    sel = (key0 > tk) | ((key0 == tk) & (li <= tif))
    oh_ref[...] = sel.astype(jnp.float32)


def _select(logits):
    idx, oh = pl.pallas_call(
        _select_body,
        out_shape=[jax.ShapeDtypeStruct((B, K), jnp.int32),
                   jax.ShapeDtypeStruct((NW, CHW), jnp.float32)],
    )(logits.reshape(NW, CHW))
    return idx, oh.reshape(B, N_INPUT)


def kernel(x, W1, b1, ln_g, ln_b, W2, b2, neuron_keys, k_input):
    gc = _maxpool(x)
    h = gc @ W1 + b1
    h = jax.nn.gelu(h, approximate=False)
    mu = jnp.mean(h, axis=-1, keepdims=True)
    var = jnp.mean((h - mu) ** 2, axis=-1, keepdims=True)
    h = (h - mu) / jnp.sqrt(var + 1e-5) * ln_g + ln_b
    query = h @ W2 + b2
    logits = _logits(query, neuron_keys)
    input_idx, one_hot = _select(logits)
    return (input_idx, one_hot)


# speedup vs baseline: 2.0235x; 1.1276x over previous
import jax
import jax.numpy as jnp
import numpy as np
from jax.experimental import pallas as pl
from jax.experimental.pallas import tpu as pltpu

B, S, D_MODEL = 4, 8192, 1024
D_ROUTING = 256
N_INPUT = 32768
K = 2048
S_CHUNK = 1024
N_CHUNK = 4096
LOGN = 15

_M7F = np.int32(0x7FFFFFFF)


def _maxpool_body(x_ref, o_ref):
    s = pl.program_id(1)
    m = jnp.max(x_ref[0], axis=0, keepdims=True)[None]

    @pl.when(s == 0)
    def _init():
        o_ref[...] = m

    @pl.when(s != 0)
    def _acc():
        o_ref[...] = jnp.maximum(o_ref[...], m)


def _maxpool(x):
    out = pl.pallas_call(
        _maxpool_body,
        grid=(B, S // S_CHUNK),
        in_specs=[pl.BlockSpec((1, S_CHUNK, D_MODEL), lambda b, s: (b, s, 0))],
        out_specs=pl.BlockSpec((1, 1, D_MODEL), lambda b, s: (b, 0, 0)),
        out_shape=jax.ShapeDtypeStruct((B, 1, D_MODEL), jnp.float32),
        compiler_params=pltpu.CompilerParams(
            dimension_semantics=("parallel", "arbitrary")),
    )(x)
    return out.reshape(B, D_MODEL)


def _logits_body(q_ref, nk_ref, o_ref):
    o_ref[...] = jax.lax.dot_general(
        q_ref[...], nk_ref[...],
        dimension_numbers=(((1,), (1,)), ((), ())),
        preferred_element_type=jnp.float32,
    ) * 0.0625


def _logits(query, nk):
    return pl.pallas_call(
        _logits_body,
        grid=(N_INPUT // N_CHUNK,),
        in_specs=[pl.BlockSpec((B, D_ROUTING), lambda n: (0, 0)),
                  pl.BlockSpec((N_CHUNK, D_ROUTING), lambda n: (n, 0))],
        out_specs=pl.BlockSpec((B, N_CHUNK), lambda n: (0, n)),
        out_shape=jax.ShapeDtypeStruct((B, N_INPUT), jnp.float32),
    )(query, nk)


NW = 64
CHW = 2048
LOGC = 11


def _select_body(v_ref, idx_ref, oh_ref):
    v = v_ref[...]
    bits = jax.lax.bitcast_convert_type(v, jnp.int32)
    key0 = bits ^ (jax.lax.shift_right_arithmetic(bits, 31) & _M7F)
    ti_ = jax.lax.broadcasted_iota(jnp.int32, (NW, CHW), 1)
    ci_ = jax.lax.broadcasted_iota(jnp.int32, (NW, CHW), 0) & np.int32(15)
    li = ci_ * np.int32(CHW) + ti_
    kcur = key0
    icur = li

    for kk in range(1, LOGC + 1):
        for j in range(kk - 1, -1, -1):
            d = 1 << j
            kup = pltpu.roll(kcur, CHW - d, 1)
            kdn = pltpu.roll(kcur, d, 1)
            iup = pltpu.roll(icur, CHW - d, 1)
            idn = pltpu.roll(icur, d, 1)
            low = (ti_ & np.int32(d)) == 0
            kp = jnp.where(low, kup, kdn)
            ip = jnp.where(low, iup, idn)
            if kk < LOGC:
                asc = (ti_ & np.int32(1 << kk)) == 0
            else:
                asc = (ci_ & np.int32(1)) == 0
            pref = (kcur > kp) | ((kcur == kp) & (icur < ip))
            keep = (low == asc) == pref
            kcur = jnp.where(keep, kcur, kp)
            icur = jnp.where(keep, icur, ip)

    rr, cpr = NW, 16
    while cpr > 1:
        kdn = pltpu.roll(kcur, 1, 0)
        kup = pltpu.roll(kcur, rr - 1, 0)
        idn = pltpu.roll(icur, 1, 0)
        iup = pltpu.roll(icur, rr - 1, 0)
        low = (ci_ & np.int32(1)) == 0
        kp = jnp.where(low, kup, kdn)
        ip = jnp.where(low, iup, idn)
        pref = (kcur > kp) | ((kcur == kp) & (icur < ip))
        keep = low == pref
        kcur = jnp.where(keep, kcur, kp)
        icur = jnp.where(keep, icur, ip)
        kcur = kcur.reshape(rr // 2, 2, CHW)[:, 0, :]
        icur = icur.reshape(rr // 2, 2, CHW)[:, 0, :]
        rr //= 2
        cpr //= 2
        ti_ = jax.lax.broadcasted_iota(jnp.int32, (rr, CHW), 1)
        ci_ = (jax.lax.broadcasted_iota(jnp.int32, (rr, CHW), 0)
               & np.int32(cpr - 1))
        for j in range(LOGC - 1, -1, -1):
            d = 1 << j
            kup = pltpu.roll(kcur, CHW - d, 1)
            kdn = pltpu.roll(kcur, d, 1)
            iup = pltpu.roll(icur, CHW - d, 1)
            idn = pltpu.roll(icur, d, 1)
            low = (ti_ & np.int32(d)) == 0
            kp = jnp.where(low, kup, kdn)
            ip = jnp.where(low, iup, idn)
            asc = (ci_ & np.int32(1)) == 0
            pref = (kcur > kp) | ((kcur == kp) & (icur < ip))
            keep = (low == asc) == pref
            kcur = jnp.where(keep, kcur, kp)
            icur = jnp.where(keep, icur, ip)

    idx_ref[...] = icur
    tk = jnp.concatenate(
        [jnp.broadcast_to(kcur[b2:b2 + 1, K - 1:K], (16, 1))
         for b2 in range(B)], axis=0)
    tif = jnp.concatenate(
        [jnp.broadcast_to(icur[b2:b2 + 1, K - 1:K], (16, 1))
         for b2 in range(B)], axis=0)
    sel = (key0 > tk) | ((key0 == tk) & (li <= tif))
    oh_ref[...] = sel.astype(jnp.float32)


def _select(logits):
    idx, oh = pl.pallas_call(
        _select_body,
        out_shape=[jax.ShapeDtypeStruct((B, K), jnp.int32),
                   jax.ShapeDtypeStruct((NW, CHW), jnp.float32)],
    )(logits.reshape(NW, CHW))
    return idx, oh.reshape(B, N_INPUT)


def kernel(x, W1, b1, ln_g, ln_b, W2, b2, neuron_keys, k_input):
    gc = _maxpool(x)
    h = gc @ W1 + b1
    h = jax.nn.gelu(h, approximate=False)
    mu = jnp.mean(h, axis=-1, keepdims=True)
    var = jnp.mean((h - mu) ** 2, axis=-1, keepdims=True)
    h = (h - mu) / jnp.sqrt(var + 1e-5) * ln_g + ln_b
    query = h @ W2 + b2
    logits = _logits(query, neuron_keys)
    input_idx, one_hot = _select(logits)
    return (input_idx, one_hot)


# speedup vs baseline: 2.0732x; 1.0246x over previous
import jax
import jax.numpy as jnp
import numpy as np
from jax.experimental import pallas as pl
from jax.experimental.pallas import tpu as pltpu

B, S, D_MODEL = 4, 8192, 1024
D_ROUTING = 256
N_INPUT = 32768
K = 2048
S_CHUNK = 2048
N_CHUNK = 8192
LOGN = 15

_M7F = np.int32(0x7FFFFFFF)


def _maxpool_body(x_ref, o_ref):
    s = pl.program_id(1)
    m = jnp.max(x_ref[0], axis=0, keepdims=True)[None]

    @pl.when(s == 0)
    def _init():
        o_ref[...] = m

    @pl.when(s != 0)
    def _acc():
        o_ref[...] = jnp.maximum(o_ref[...], m)


def _maxpool(x):
    out = pl.pallas_call(
        _maxpool_body,
        grid=(B, S // S_CHUNK),
        in_specs=[pl.BlockSpec((1, S_CHUNK, D_MODEL), lambda b, s: (b, s, 0))],
        out_specs=pl.BlockSpec((1, 1, D_MODEL), lambda b, s: (b, 0, 0)),
        out_shape=jax.ShapeDtypeStruct((B, 1, D_MODEL), jnp.float32),
        compiler_params=pltpu.CompilerParams(
            dimension_semantics=("parallel", "arbitrary")),
    )(x)
    return out.reshape(B, D_MODEL)


def _logits_body(q_ref, nk_ref, o_ref):
    o_ref[...] = jax.lax.dot_general(
        q_ref[...], nk_ref[...],
        dimension_numbers=(((1,), (1,)), ((), ())),
        preferred_element_type=jnp.float32,
    ) * 0.0625


def _logits(query, nk):
    return pl.pallas_call(
        _logits_body,
        grid=(N_INPUT // N_CHUNK,),
        in_specs=[pl.BlockSpec((B, D_ROUTING), lambda n: (0, 0)),
                  pl.BlockSpec((N_CHUNK, D_ROUTING), lambda n: (n, 0))],
        out_specs=pl.BlockSpec((B, N_CHUNK), lambda n: (0, n)),
        out_shape=jax.ShapeDtypeStruct((B, N_INPUT), jnp.float32),
    )(query, nk)


NW = 64
CHW = 2048
LOGC = 11


def _select_body(v_ref, idx_ref, oh_ref):
    v = v_ref[...]
    bits = jax.lax.bitcast_convert_type(v, jnp.int32)
    key0 = bits ^ (jax.lax.shift_right_arithmetic(bits, 31) & _M7F)
    ti_ = jax.lax.broadcasted_iota(jnp.int32, (NW, CHW), 1)
    ci_ = jax.lax.broadcasted_iota(jnp.int32, (NW, CHW), 0) & np.int32(15)
    li = ci_ * np.int32(CHW) + ti_
    kcur = key0
    icur = li

    for kk in range(1, LOGC + 1):
        for j in range(kk - 1, -1, -1):
            d = 1 << j
            kup = pltpu.roll(kcur, CHW - d, 1)
            kdn = pltpu.roll(kcur, d, 1)
            iup = pltpu.roll(icur, CHW - d, 1)
            idn = pltpu.roll(icur, d, 1)
            low = (ti_ & np.int32(d)) == 0
            kp = jnp.where(low, kup, kdn)
            ip = jnp.where(low, iup, idn)
            if kk < LOGC:
                asc = (ti_ & np.int32(1 << kk)) == 0
            else:
                asc = (ci_ & np.int32(1)) == 0
            pref = (kcur > kp) | ((kcur == kp) & (icur < ip))
            keep = (low == asc) == pref
            kcur = jnp.where(keep, kcur, kp)
            icur = jnp.where(keep, icur, ip)

    rr, cpr = NW, 16
    while cpr > 1:
        kdn = pltpu.roll(kcur, 1, 0)
        kup = pltpu.roll(kcur, rr - 1, 0)
        idn = pltpu.roll(icur, 1, 0)
        iup = pltpu.roll(icur, rr - 1, 0)
        low = (ci_ & np.int32(1)) == 0
        kp = jnp.where(low, kup, kdn)
        ip = jnp.where(low, iup, idn)
        pref = (kcur > kp) | ((kcur == kp) & (icur < ip))
        keep = low == pref
        kcur = jnp.where(keep, kcur, kp)
        icur = jnp.where(keep, icur, ip)
        kcur = kcur.reshape(rr // 2, 2, CHW)[:, 0, :]
        icur = icur.reshape(rr // 2, 2, CHW)[:, 0, :]
        rr //= 2
        cpr //= 2
        ti_ = jax.lax.broadcasted_iota(jnp.int32, (rr, CHW), 1)
        ci_ = (jax.lax.broadcasted_iota(jnp.int32, (rr, CHW), 0)
               & np.int32(cpr - 1))
        for j in range(LOGC - 1, -1, -1):
            d = 1 << j
            kup = pltpu.roll(kcur, CHW - d, 1)
            kdn = pltpu.roll(kcur, d, 1)
            iup = pltpu.roll(icur, CHW - d, 1)
            idn = pltpu.roll(icur, d, 1)
            low = (ti_ & np.int32(d)) == 0
            kp = jnp.where(low, kup, kdn)
            ip = jnp.where(low, iup, idn)
            asc = (ci_ & np.int32(1)) == 0
            pref = (kcur > kp) | ((kcur == kp) & (icur < ip))
            keep = (low == asc) == pref
            kcur = jnp.where(keep, kcur, kp)
            icur = jnp.where(keep, icur, ip)

    idx_ref[...] = icur
    tk = jnp.concatenate(
        [jnp.broadcast_to(kcur[b2:b2 + 1, K - 1:K], (16, 1))
         for b2 in range(B)], axis=0)
    tif = jnp.concatenate(
        [jnp.broadcast_to(icur[b2:b2 + 1, K - 1:K], (16, 1))
         for b2 in range(B)], axis=0)
    sel = (key0 > tk) | ((key0 == tk) & (li <= tif))
    oh_ref[...] = sel.astype(jnp.float32)


def _select(logits):
    idx, oh = pl.pallas_call(
        _select_body,
        out_shape=[jax.ShapeDtypeStruct((B, K), jnp.int32),
                   jax.ShapeDtypeStruct((NW, CHW), jnp.float32)],
    )(logits.reshape(NW, CHW))
    return idx, oh.reshape(B, N_INPUT)


def kernel(x, W1, b1, ln_g, ln_b, W2, b2, neuron_keys, k_input):
    gc = _maxpool(x)
    h = gc @ W1 + b1
    h = jax.nn.gelu(h, approximate=False)
    mu = jnp.mean(h, axis=-1, keepdims=True)
    var = jnp.mean((h - mu) ** 2, axis=-1, keepdims=True)
    h = (h - mu) / jnp.sqrt(var + 1e-5) * ln_g + ln_b
    query = h @ W2 + b2
    logits = _logits(query, neuron_keys)
    input_idx, one_hot = _select(logits)
    return (input_idx, one_hot)
